# Initial kernel scaffold; baseline (speedup 1.0000x reference)
import jax
import jax.numpy as jnp
from jax.experimental import pallas as pl


def kernel(x, edge_index, Wr, br, Wu, bu, Wc, bc, ln_gamma, ln_beta, Wout, bout):
    raise NotImplementedError("write your pallas kernel here")


# speedup vs baseline: 3.9386x; 3.9386x over previous
import functools

import jax
import jax.numpy as jnp
from jax import lax
from jax.experimental import pallas as pl
from jax.experimental.pallas import tpu as pltpu
from jax.experimental.pallas import tpu_sc as plsc

N = 10000
T = 12
D = 128
H = 128
E = 160000

NC = 2
NS = 16
CH = 128
EPC = E
HALF = 5120
ACCL = 5248
ZS = ACCL // NS
CS = HALF // NS
OUTR = 2 * HALF
DUMLOC = 5184


def _make_prop(M, dirs):
  mesh = plsc.VectorSubcoreMesh(
      core_axis_name="c", subcore_axis_name="s", num_cores=NC, num_subcores=NS)
  out_type = [jax.ShapeDtypeStruct((OUTR, 128), jnp.float32) for _ in range(M)]
  scratch = [
      pltpu.VMEM((16,), jnp.int32),
      pltpu.VMEM((CH,), jnp.int32),
      pltpu.VMEM((CH,), jnp.int32),
      pltpu.VMEM((CH, 128), jnp.float32),
      pltpu.VMEM((ZS, 128), jnp.float32),
      pltpu.VMEM_SHARED((ACCL, 128), jnp.float32),
      pltpu.SemaphoreType.DMA,
  ]

  @functools.partial(pl.kernel, mesh=mesh, out_type=out_type,
                     scratch_types=scratch, name=f"sc_prop_m{M}")
  def kfn(*refs):
    zs = refs[:M]
    idxg4, idxs4, nch_hbm, zrow = refs[M:M + 4]
    outs = refs[M + 4: M + 4 + M]
    nch_v, idxg_v, idxs_v, rows_v, zero_v, acc, sem = refs[M + 4 + M:]
    cid = lax.axis_index("c")
    sid = lax.axis_index("s")
    pltpu.sync_copy(zrow, zero_v)
    pltpu.sync_copy(nch_hbm, nch_v)
    nchv = nch_v[...]
    for m in range(M):
      nch = jnp.where(cid == 0, nchv[2 * dirs[m]], nchv[2 * dirs[m] + 1])
      n_w = jnp.maximum(0, (nch - sid + NS - 1) // NS)
      pltpu.sync_copy(zero_v, acc.at[pl.ds(sid * ZS, ZS)])
      plsc.subcore_barrier()

      def chunk(i, carry):
        base = (sid + i * NS) * CH
        pltpu.sync_copy(idxg4.at[dirs[m], cid, pl.ds(base, CH)], idxg_v)
        pltpu.sync_copy(idxs4.at[dirs[m], cid, pl.ds(base, CH)], idxs_v)
        pltpu.async_copy(zs[m].at[idxg_v], rows_v, sem).wait()
        pltpu.sync_copy(rows_v, acc.at[idxs_v], add=True)
        return carry

      lax.fori_loop(0, n_w, chunk, 0)
      plsc.subcore_barrier()
      pltpu.sync_copy(acc.at[pl.ds(sid * CS, CS)],
                      outs[m].at[pl.ds(cid * HALF + sid * CS, CS)])
      plsc.subcore_barrier()

  return kfn


_PROP_KERNELS = {}


def _prop(zs, idxg4, idxs4, nch, dirs):
  key = (len(zs), dirs)
  if key not in _PROP_KERNELS:
    _PROP_KERNELS[key] = _make_prop(len(zs), dirs)
  zrow = jnp.zeros((ZS, 128), jnp.float32)
  outs = _PROP_KERNELS[key](*zs, idxg4, idxs4, nch, zrow)
  if len(zs) == 1:
    outs = (outs,)
  return [o[:N] for o in outs]


def _bucket_edges(gather_idx, scatter_idx):
  owner1 = (scatter_idx >= HALF).astype(jnp.int32)
  cnt0 = E - jnp.sum(owner1)
  c0 = jnp.cumsum(1 - owner1) - 1
  c1 = jnp.cumsum(owner1) - 1
  pos = jnp.where(owner1 == 0, c0, cnt0 + c1)
  g_sorted = jnp.zeros((E,), jnp.int32).at[pos].set(gather_idx)
  s_sorted = jnp.zeros((E,), jnp.int32).at[pos].set(scatter_idx)
  j = jnp.arange(EPC, dtype=jnp.int32)
  g0 = jnp.where(j < cnt0, g_sorted, 0)
  s0 = jnp.where(j < cnt0, s_sorted, DUMLOC)
  j1 = jnp.clip(cnt0 + j, 0, E - 1)
  cnt1 = E - cnt0
  g1 = jnp.where(j < cnt1, g_sorted[j1], 0)
  s1 = jnp.where(j < cnt1, s_sorted[j1] - HALF, DUMLOC)
  nch = jnp.stack([(cnt0 + CH - 1) // CH, (cnt1 + CH - 1) // CH])
  return jnp.stack([g0, g1]), jnp.stack([s0, s1]), nch


def kernel(x, edge_index, Wr, br, Wu, bu, Wc, bc, ln_gamma, ln_beta, Wout, bout):
  src = edge_index[0]
  dst = edge_index[1]
  ones = jnp.ones((E,), jnp.float32)
  deg_out = jax.ops.segment_sum(ones, src, num_segments=N)
  deg_in = jax.ops.segment_sum(ones, dst, num_segments=N)
  dinv_out = jnp.where(deg_out > 0, 1.0 / deg_out, 0.0)[:, None]
  dinv_in = jnp.where(deg_in > 0, 1.0 / deg_in, 0.0)[:, None]

  g0, s0, nch0 = _bucket_edges(dst, src)
  g1, s1, nch1 = _bucket_edges(src, dst)
  idxg4 = jnp.stack([g0, g1])
  idxs4 = jnp.stack([s0, s1])
  nch = jnp.zeros((16,), jnp.int32).at[:4].set(
      jnp.concatenate([nch0, nch1]).astype(jnp.int32))

  def split(W):
    Wx = [W[s * 256: s * 256 + 128] for s in range(5)]
    Wh = [W[s * 256 + 128: s * 256 + 256] for s in range(5)]
    return Wx, Wh

  Wrx, Wrh = split(Wr)
  Wux, Wuh = split(Wu)
  Wcx, Wch = split(Wc)
  Wxcat = jnp.concatenate(
      [jnp.concatenate([Wrx[s], Wux[s], Wcx[s]], axis=1) for s in range(5)],
      axis=0)
  Whru = jnp.concatenate(
      [jnp.concatenate([Wrh[s], Wuh[s]], axis=1) for s in range(5)],
      axis=0)
  Whc = jnp.concatenate([Wch[s] for s in range(5)], axis=0)
  bru = jnp.concatenate([br, bu])

  def prop(zs, dirs):
    return _prop(zs, idxg4, idxs4, nch, dirs)

  xT = [x[:, t, :] for t in range(T)]
  dirsA = (0,) * T + (1,) * T
  hop1 = prop(xT + xT, dirsA)
  x1 = [hop1[t] * dinv_out for t in range(T)]
  x3 = [hop1[T + t] * dinv_in for t in range(T)]
  hop2 = prop(x1 + x3, dirsA)
  x2 = [hop2[t] * dinv_out for t in range(T)]
  x4 = [hop2[T + t] * dinv_in for t in range(T)]

  xcat = jnp.concatenate(
      [jnp.stack(a, axis=1) for a in (xT, x1, x2, x3, x4)], axis=-1)
  G = (xcat.reshape(N * T, 5 * D) @ Wxcat).reshape(N, T, 3 * H)

  h = jnp.zeros((N, H), jnp.float32)
  outs = []
  for t in range(T):
    y1, y3 = prop([h, h], (0, 1))
    h1, h3 = y1 * dinv_out, y3 * dinv_in
    y2, y4 = prop([h1, h3], (0, 1))
    h2, h4 = y2 * dinv_out, y4 * dinv_in
    hp = jnp.concatenate([h, h1, h2, h3, h4], axis=1)
    ru = jax.nn.sigmoid(G[:, t, :2 * H] + hp @ Whru + bru)
    r, u = ru[:, :H], ru[:, H:]
    rh = r * h
    y1, y3 = prop([rh, rh], (0, 1))
    g1_, g3_ = y1 * dinv_out, y3 * dinv_in
    y2, y4 = prop([g1_, g3_], (0, 1))
    g2_, g4_ = y2 * dinv_out, y4 * dinv_in
    gp = jnp.concatenate([rh, g1_, g2_, g3_, g4_], axis=1)
    c = jnp.tanh(G[:, t, 2 * H:] + gp @ Whc + bc)
    h = u * h + (1.0 - u) * c
    outs.append(h)

  seq = jnp.stack(outs, axis=1)
  feats = jnp.concatenate(
      [seq[:, -1], jnp.mean(seq, axis=1), jnp.max(seq, axis=1)], axis=1)
  mu = jnp.mean(feats, axis=-1, keepdims=True)
  var = jnp.var(feats, axis=-1, keepdims=True)
  normed = (feats - mu) / jnp.sqrt(var + 1e-5) * ln_gamma + ln_beta
  return (normed @ Wout + bout)[:, 0]
